# baseline (speedup 1.0000x reference)
import jax
import jax.numpy as jnp
from jax.experimental import pallas as pl

_EPS = 1e-5


def _head_kernel(x_ref, w_ref, t_ref, o_ref):
    x = x_ref[0]
    w = w_ref[...]
    t = t_ref[...]
    y = jax.lax.dot_general(
        x, w, (((0,), (0,)), ((), ())),
        preferred_element_type=jnp.float32,
    )
    o_ref[0] = jnp.maximum(y + t, 0.0)


def kernel(xyz, params):
    W, bb, g, be = params["head"][0]
    s = g / jnp.sqrt(1.0 + _EPS)
    wt = (W * s[:, None]).T
    t = (bb * s + be)[None, :]
    B, C, N = xyz.shape
    O = W.shape[0]
    return pl.pallas_call(
        _head_kernel,
        grid=(B,),
        in_specs=[
            pl.BlockSpec((1, C, N), lambda b: (b, 0, 0)),
            pl.BlockSpec((C, O), lambda b: (0, 0)),
            pl.BlockSpec((1, O), lambda b: (0, 0)),
        ],
        out_specs=pl.BlockSpec((1, N, O), lambda b: (b, 0, 0)),
        out_shape=jax.ShapeDtypeStruct((B, N, O), xyz.dtype),
    )(xyz, wt, t)
